# Initial kernel scaffold; baseline (speedup 1.0000x reference)
import jax
import jax.numpy as jnp
from jax.experimental import pallas as pl


def kernel(user_emb, item_emb, edge_index, noise):
    raise NotImplementedError("write your pallas kernel here")


# speedup vs baseline: 8.4262x; 8.4262x over previous
import functools

import jax
import jax.numpy as jnp
from jax import lax
from jax.experimental import pallas as pl
from jax.experimental.pallas import tpu as pltpu
from jax.experimental.pallas import tpu_sc as plsc

U_N = 25000
N = 50000
E = 600000
D = 64
DH = 32
EPS = 0.2

NC, NS, L = 2, 16, 16
NW = NC * NS

EP = 1200128
CH = 128
EPT = EP // NS
NCHUNK = EPT // CH
EPW = EP // NW
NPAD = 51200
JUNK = N
OPT = 3128
NOUT = NS * OPT

BN = 5000
GN = N // BN
BP = 6400
GP = NPAD // BP

_mesh = plsc.VectorSubcoreMesh(
    core_axis_name="c", subcore_axis_name="s", num_cores=NC, num_subcores=NS
)
_sc_params = pltpu.CompilerParams(
    needs_layout_passes=False, use_tc_tiling_on_sc=False
)


@functools.partial(
    pl.kernel,
    out_type=jax.ShapeDtypeStruct((NW, NPAD), jnp.float32),
    mesh=_mesh,
    scratch_types=[
        pltpu.VMEM((NPAD,), jnp.float32),
        pltpu.VMEM((EPW,), jnp.int32),
    ],
    compiler_params=_sc_params,
)
def _sc_deg(src_hbm, zeros_hbm, out_hbm, degl, idxb):
    c = lax.axis_index("c")
    s = lax.axis_index("s")
    wid = c * NS + s
    pltpu.sync_copy(zeros_hbm, degl)
    pltpu.sync_copy(src_hbm.at[pl.ds(wid * EPW, EPW)], idxb)
    ones = jnp.full((L,), 1.0, jnp.float32)

    @pl.loop(0, EPW // L)
    def _(i):
        v = idxb[pl.ds(i * L, L)]
        plsc.addupdate_scatter(degl, [v], ones)

    pltpu.sync_copy(degl, out_hbm.at[wid])


@functools.partial(
    pl.kernel,
    out_type=jax.ShapeDtypeStruct((2, NOUT, DH), jnp.float32),
    mesh=_mesh,
    scratch_types=[
        pltpu.VMEM((CH,), jnp.int32),
        pltpu.VMEM((CH,), jnp.int32),
        pltpu.VMEM((CH, DH), jnp.float32),
        pltpu.VMEM_SHARED((NPAD, DH), jnp.float32),
        pltpu.SemaphoreType.DMA,
    ],
    compiler_params=_sc_params,
)
def _sc_prop(tbl_hbm, src_hbm, dst_hbm, out_hbm, srcb, dstb, rows, accs, sem):
    c = lax.axis_index("c")
    s = lax.axis_index("s")

    zeros16 = jnp.zeros((L,), jnp.float32)

    @pl.loop(0, CH)
    def _(r):
        for q in range(DH // L):
            rows[r, pl.ds(q * L, L)] = zeros16

    zpt = NPAD // NS

    @pl.loop(0, zpt // CH)
    def _(z):
        pltpu.sync_copy(rows, accs.at[pl.ds(s * zpt + z * CH, CH)])

    plsc.subcore_barrier()

    base = s * EPT
    coff = c * N

    @pl.loop(0, NCHUNK)
    def _(j):
        e0 = base + j * CH
        pltpu.sync_copy(src_hbm.at[pl.ds(e0, CH)], srcb)
        pltpu.sync_copy(dst_hbm.at[pl.ds(e0, CH)], dstb)
        for q in range(CH // L):
            v = srcb[pl.ds(q * L, L)]
            srcb[pl.ds(q * L, L)] = v + coff
        pltpu.async_copy(tbl_hbm.at[srcb], rows, sem).wait()
        pltpu.sync_copy(rows, accs.at[dstb], add=True)

    plsc.subcore_barrier()

    pltpu.sync_copy(
        accs.at[pl.ds(s * OPT, OPT)], out_hbm.at[c, pl.ds(s * OPT, OPT)]
    )


def _isd_body(deg_ref, isd_ref):
    d = jnp.sum(deg_ref[...], axis=0)
    d = jnp.maximum(d, 1.0)
    isd_ref[...] = lax.rsqrt(d)[:, None]


_tc_isd = pl.pallas_call(
    _isd_body,
    grid=(GP,),
    in_specs=[pl.BlockSpec((NW, BP), lambda b: (0, b))],
    out_specs=pl.BlockSpec((BP, 1), lambda b: (b, 0)),
    out_shape=jax.ShapeDtypeStruct((NPAD, 1), jnp.float32),
)


def _tilde0_body(ue_ref, ie_ref, isd_ref, tbl_ref):
    b = pl.program_id(0)
    ego = jnp.where(b < GN // 2, ue_ref[...], ie_ref[...])
    t = ego * isd_ref[...]
    tbl_ref[...] = t.reshape(BN, 2, DH).swapaxes(0, 1)


_tc_tilde0 = pl.pallas_call(
    _tilde0_body,
    grid=(GN,),
    in_specs=[
        pl.BlockSpec((BN, D), lambda b: (jnp.minimum(b, GN // 2 - 1), 0)),
        pl.BlockSpec((BN, D), lambda b: (jnp.maximum(b - GN // 2, 0), 0)),
        pl.BlockSpec((BN, 1), lambda b: (b, 0)),
    ],
    out_specs=pl.BlockSpec((2, BN, DH), lambda b: (0, b, 0)),
    out_shape=jax.ShapeDtypeStruct((2, N, DH), jnp.float32),
)


def _make_dense(layer):
    def body(*refs):
        if layer == 1:
            alo, ahi, isd, nz, ego_o, tbl_o = refs
        elif layer == 2:
            alo, ahi, isd, nz, prev, sum_o, tbl_o = refs
        else:
            alo, ahi, isd, nz, prev, fin_o = refs
        acc = jnp.concatenate([alo[0], ahi[0]], axis=-1)
        isdv = isd[...]
        ego = acc * isdv
        r = nz[...]
        nrm = jnp.sqrt(jnp.sum(r * r, axis=-1, keepdims=True))
        rn = r / (nrm + 1e-12)
        ego = ego + jnp.sign(ego) * rn * EPS
        if layer == 1:
            ego_o[...] = ego
        elif layer == 2:
            sum_o[...] = prev[...] + ego
        else:
            fin_o[...] = (prev[...] + ego) / 3.0
        if layer < 3:
            t = ego * isdv
            tbl_o[...] = t.reshape(BN, 2, DH).swapaxes(0, 1)

    n_spec = pl.BlockSpec((BN, D), lambda b: (b, 0))
    in_specs = [
        pl.BlockSpec((1, BN, DH), lambda b: (0, b, 0)),
        pl.BlockSpec((1, BN, DH), lambda b: (1, b, 0)),
        pl.BlockSpec((BN, 1), lambda b: (b, 0)),
        n_spec,
    ]
    if layer > 1:
        in_specs.append(n_spec)
    tbl_spec = pl.BlockSpec((2, BN, DH), lambda b: (0, b, 0))
    tbl_shape = jax.ShapeDtypeStruct((2, N, DH), jnp.float32)
    n_shape = jax.ShapeDtypeStruct((N, D), jnp.float32)
    if layer < 3:
        out_specs = (n_spec, tbl_spec)
        out_shape = (n_shape, tbl_shape)
    else:
        out_specs = n_spec
        out_shape = n_shape
    return pl.pallas_call(
        body,
        grid=(GN,),
        in_specs=in_specs,
        out_specs=out_specs,
        out_shape=out_shape,
    )


_tc_dense1 = _make_dense(1)
_tc_dense2 = _make_dense(2)
_tc_dense3 = _make_dense(3)


def kernel(user_emb, item_emb, edge_index, noise):
    u = edge_index[0].astype(jnp.int32)
    it = edge_index[1].astype(jnp.int32) + U_N
    npd = EP - 2 * E
    src = jnp.concatenate([u, it, jnp.zeros((npd,), jnp.int32)])
    dst = jnp.concatenate([it, u, jnp.full((npd,), JUNK, jnp.int32)])
    src_deg = jnp.concatenate([u, it, jnp.full((npd,), JUNK, jnp.int32)])
    zerosN = jnp.zeros((NPAD,), jnp.float32)

    degp = _sc_deg(src_deg, zerosN)
    isd = _tc_isd(degp)
    tbl = _tc_tilde0(user_emb, item_emb, isd).reshape(2 * N, DH)
    acc = _sc_prop(tbl, src, dst)
    ego1, t1 = _tc_dense1(acc, acc, isd, noise[0])
    acc = _sc_prop(t1.reshape(2 * N, DH), src, dst)
    s2, t2 = _tc_dense2(acc, acc, isd, noise[1], ego1)
    acc = _sc_prop(t2.reshape(2 * N, DH), src, dst)
    fin = _tc_dense3(acc, acc, isd, noise[2], s2)
    return jnp.stack([fin, ego1], axis=0)
